# baseline (speedup 1.0000x reference)
import functools

import jax
import jax.numpy as jnp
from jax import lax
from jax.experimental import pallas as pl
from jax.experimental.pallas import tpu as pltpu
from jax.experimental.pallas import tpu_sc as plsc

B = 4096
F = 16384
NW = 32
RPW = B // NW
NCHUNK = RPW // 2


def _sc_scores_body(feat_hbm, w_hbm, out_hbm, w_v, buf_a, buf_b,
                    scores_v, sem_a, sem_b):
    wid = lax.axis_index("s") * 2 + lax.axis_index("c")
    base = wid * RPW
    pltpu.sync_copy(w_hbm, w_v)
    pltpu.async_copy(feat_hbm.at[pl.ds(base, 2)], buf_a, sem_a)
    pltpu.async_copy(feat_hbm.at[pl.ds(base + 2, 2)], buf_b, sem_b)

    def dot2(buf):
        def body(j, carry):
            a0, a1 = carry
            w = w_v[pl.ds(j * 16, 16)]
            f0 = buf[0, pl.ds(j * 16, 16)]
            f1 = buf[1, pl.ds(j * 16, 16)]
            return (a0 + f0 * w, a1 + f1 * w)

        z = jnp.zeros((16,), jnp.float32)
        return lax.fori_loop(0, F // 16, body, (z, z), unroll=8)

    lane = lax.iota(jnp.int32, 16)

    def outer(c2, svec):
        for par, buf, sem in ((0, buf_a, sem_a), (1, buf_b, sem_b)):
            c = 2 * c2 + par
            pltpu.make_async_copy(feat_hbm.at[pl.ds(base, 2)], buf, sem).wait()
            a0, a1 = dot2(buf)
            for o, acc in ((0, a0), (1, a1)):
                r = 2 * c + o
                svec = jnp.where(lane == r % 16, jnp.sum(acc), svec)

            @pl.when(c + 2 < NCHUNK)
            def _():
                pltpu.async_copy(
                    feat_hbm.at[pl.ds(base + (c + 2) * 2, 2)], buf, sem)

        @pl.when(c2 % 4 == 3)
        def _():
            scores_v[pl.ds((c2 // 4) * 16, 16)] = svec

        return svec

    lax.fori_loop(0, NCHUNK // 2, outer, jnp.zeros((16,), jnp.float32))
    pltpu.sync_copy(scores_v, out_hbm.at[pl.ds(base, RPW)])


def _sc_scores(feat, w_flat):
    return pl.kernel(
        _sc_scores_body,
        out_type=jax.ShapeDtypeStruct((B,), jnp.float32),
        mesh=plsc.VectorSubcoreMesh(core_axis_name="c", subcore_axis_name="s"),
        compiler_params=pltpu.CompilerParams(needs_layout_passes=False),
        scratch_types=[
            pltpu.VMEM((F,), jnp.float32),
            pltpu.VMEM((2, F), jnp.float32),
            pltpu.VMEM((2, F), jnp.float32),
            pltpu.VMEM((RPW,), jnp.float32),
            pltpu.SemaphoreType.DMA,
            pltpu.SemaphoreType.DMA,
        ],
    )(feat, w_flat)


def _softmax_body(s_ref, out_ref):
    s = s_ref[...]
    m = jnp.max(s)
    e = jnp.exp(s - m)
    out_ref[...] = e / jnp.sum(e)


def _softmax(scores_2d):
    return pl.pallas_call(
        _softmax_body,
        out_shape=jax.ShapeDtypeStruct((1, B), jnp.float32),
    )(scores_2d)


def kernel(feature_vector, linear):
    scores = _sc_scores(feature_vector, linear.reshape(F))
    probs = _softmax(scores.reshape(1, B))
    return probs.reshape(1, B, 1)


# speedup vs baseline: 1.4217x; 1.4217x over previous
import jax
import jax.numpy as jnp
from jax import lax
from jax.experimental import pallas as pl
from jax.experimental.pallas import tpu as pltpu
from jax.experimental.pallas import tpu_sc as plsc

B = 4096
F = 16384
NW = 32
B_SC = 1536
B_TC = B - B_SC
RPW = B_SC // NW
NCHUNK = RPW // 2
BR = 256


def _sc_scores_body(feat_hbm, w_hbm, out_hbm, w_v, buf_a, buf_b,
                    scores_v, sem_a, sem_b):
    wid = lax.axis_index("s") * 2 + lax.axis_index("c")
    base = B_TC + wid * RPW
    pltpu.sync_copy(w_hbm, w_v)
    pltpu.async_copy(feat_hbm.at[pl.ds(base, 2)], buf_a, sem_a)
    pltpu.async_copy(feat_hbm.at[pl.ds(base + 2, 2)], buf_b, sem_b)

    def dot2(buf):
        def body(j, carry):
            a0, a1 = carry
            w = w_v[pl.ds(j * 16, 16)]
            f0 = buf[0, pl.ds(j * 16, 16)]
            f1 = buf[1, pl.ds(j * 16, 16)]
            return (a0 + f0 * w, a1 + f1 * w)

        z = jnp.zeros((16,), jnp.float32)
        return lax.fori_loop(0, F // 16, body, (z, z), unroll=8)

    lane = lax.iota(jnp.int32, 16)

    def outer(c2, svec):
        for par, buf, sem in ((0, buf_a, sem_a), (1, buf_b, sem_b)):
            c = 2 * c2 + par
            pltpu.make_async_copy(feat_hbm.at[pl.ds(base, 2)], buf, sem).wait()
            a0, a1 = dot2(buf)
            for o, acc in ((0, a0), (1, a1)):
                r = 2 * c + o
                svec = jnp.where(lane == r % 16, jnp.sum(acc), svec)

            @pl.when(c + 2 < NCHUNK)
            def _():
                pltpu.async_copy(
                    feat_hbm.at[pl.ds(base + (c + 2) * 2, 2)], buf, sem)

        @pl.when(c2 % 4 == 3)
        def _():
            scores_v[pl.ds((c2 // 4) * 16, 16)] = svec

        return svec

    lax.fori_loop(0, NCHUNK // 2, outer, jnp.zeros((16,), jnp.float32))
    pltpu.sync_copy(scores_v, out_hbm.at[pl.ds(wid * RPW, RPW)])


def _sc_scores(feat, w_flat):
    return pl.kernel(
        _sc_scores_body,
        out_type=jax.ShapeDtypeStruct((B_SC,), jnp.float32),
        mesh=plsc.VectorSubcoreMesh(core_axis_name="c", subcore_axis_name="s"),
        compiler_params=pltpu.CompilerParams(needs_layout_passes=False),
        scratch_types=[
            pltpu.VMEM((F,), jnp.float32),
            pltpu.VMEM((2, F), jnp.float32),
            pltpu.VMEM((2, F), jnp.float32),
            pltpu.VMEM((RPW,), jnp.float32),
            pltpu.SemaphoreType.DMA,
            pltpu.SemaphoreType.DMA,
        ],
    )(feat, w_flat)


def _tc_scores_body(feat_ref, w_ref, out_ref):
    out_ref[...] = jnp.sum(feat_ref[...] * w_ref[...], axis=1)[None, :]


def _tc_scores(feat, w_row):
    return pl.pallas_call(
        _tc_scores_body,
        grid=(B_TC // BR,),
        in_specs=[
            pl.BlockSpec((BR, F), lambda i: (i, 0)),
            pl.BlockSpec((1, F), lambda i: (0, 0)),
        ],
        out_specs=pl.BlockSpec((1, BR), lambda i: (0, i)),
        out_shape=jax.ShapeDtypeStruct((1, B_TC), jnp.float32),
    )(feat, w_row)


def _softmax_body(a_ref, b_ref, out_ref):
    s = jnp.concatenate([a_ref[...], b_ref[...]], axis=1)
    m = jnp.max(s)
    e = jnp.exp(s - m)
    out_ref[...] = e / jnp.sum(e)


def _softmax(scores_tc, scores_sc):
    return pl.pallas_call(
        _softmax_body,
        out_shape=jax.ShapeDtypeStruct((1, B), jnp.float32),
    )(scores_tc, scores_sc)


def kernel(feature_vector, linear):
    scores_sc = _sc_scores(feature_vector, linear.reshape(F))
    scores_tc = _tc_scores(feature_vector, linear.reshape(1, F))
    probs = _softmax(scores_tc, scores_sc.reshape(1, B_SC))
    return probs.reshape(1, B, 1)
